# Initial kernel scaffold; baseline (speedup 1.0000x reference)
import jax
import jax.numpy as jnp
from jax.experimental import pallas as pl


def kernel(idx, A_param):
    raise NotImplementedError("write your pallas kernel here")


# speedup vs baseline: 13.5855x; 13.5855x over previous
import functools

import jax
import jax.numpy as jnp
from jax import lax
from jax.experimental import pallas as pl

N = 10000
K = 32
BLOCK_ROWS = 200
BITS_STEPS = 31
TOP_BITS = 0x7F800000


def _topk_mask_kernel(a_ref, o_ref):
    x = a_ref[...]
    a = jnp.maximum(x, 0.0)
    bits = lax.bitcast_convert_type(a, jnp.int32)

    lo0 = jnp.zeros((BLOCK_ROWS, 1), jnp.int32)
    hi0 = jnp.full((BLOCK_ROWS, 1), TOP_BITS, jnp.int32)

    def body(_, carry):
        lo, hi = carry
        mid = lo + lax.shift_right_logical(hi - lo, 1)
        cnt = jnp.sum((bits >= mid).astype(jnp.int32), axis=1, keepdims=True)
        take_hi = cnt >= K
        lo = jnp.where(take_hi, mid, lo)
        hi = jnp.where(take_hi, hi, mid)
        return lo, hi

    lo, _ = lax.fori_loop(0, BITS_STEPS, body, (lo0, hi0))
    o_ref[...] = jnp.where(bits >= lo, a, 0.0)


@jax.jit
def _topk_mask(a):
    grid = (N // BLOCK_ROWS,)
    return pl.pallas_call(
        _topk_mask_kernel,
        grid=grid,
        in_specs=[pl.BlockSpec((BLOCK_ROWS, N), lambda i: (i, 0))],
        out_specs=pl.BlockSpec((BLOCK_ROWS, N), lambda i: (i, 0)),
        out_shape=jax.ShapeDtypeStruct((N, N), jnp.float32),
    )(a)


def kernel(idx, A_param):
    del idx
    return _topk_mask(A_param)


# speedup vs baseline: 15.7970x; 1.1628x over previous
import functools

import jax
import jax.numpy as jnp
from jax import lax
from jax.experimental import pallas as pl

N = 10000
K = 32
BLOCK_ROWS = 200
BITS_STEPS = 31
TOP_BITS = 0x7F800000


def _topk_mask_kernel(a_ref, o_ref):
    x = a_ref[...]
    a = jnp.maximum(x, 0.0)
    bits = lax.bitcast_convert_type(a, jnp.int32)

    lo0 = jnp.zeros((BLOCK_ROWS, 1), jnp.int32)
    hi0 = jnp.full((BLOCK_ROWS, 1), TOP_BITS, jnp.int32)
    cnt0 = jnp.full((BLOCK_ROWS, 1), N, jnp.int32)

    def cond(carry):
        i, lo, hi, cnt_lo = carry
        settled = jnp.logical_or(cnt_lo == K, hi - lo <= 1)
        return jnp.logical_and(i < BITS_STEPS, jnp.logical_not(jnp.all(settled)))

    def body(carry):
        i, lo, hi, cnt_lo = carry
        mid = lo + lax.shift_right_logical(hi - lo, 1)
        cnt = jnp.sum((bits >= mid).astype(jnp.int32), axis=1, keepdims=True)
        take_hi = cnt >= K
        lo = jnp.where(take_hi, mid, lo)
        hi = jnp.where(take_hi, hi, mid)
        cnt_lo = jnp.where(take_hi, cnt, cnt_lo)
        return i + 1, lo, hi, cnt_lo

    _, lo, _, _ = lax.while_loop(cond, body, (0, lo0, hi0, cnt0))
    o_ref[...] = jnp.where(bits >= lo, a, 0.0)


@jax.jit
def _topk_mask(a):
    grid = (N // BLOCK_ROWS,)
    return pl.pallas_call(
        _topk_mask_kernel,
        grid=grid,
        in_specs=[pl.BlockSpec((BLOCK_ROWS, N), lambda i: (i, 0))],
        out_specs=pl.BlockSpec((BLOCK_ROWS, N), lambda i: (i, 0)),
        out_shape=jax.ShapeDtypeStruct((N, N), jnp.float32),
    )(a)


def kernel(idx, A_param):
    del idx
    return _topk_mask(A_param)


# speedup vs baseline: 15.8387x; 1.0026x over previous
import functools

import jax
import jax.numpy as jnp
from jax import lax
from jax.experimental import pallas as pl

N = 10000
K = 32
BLOCK_ROWS = 200
BITS_STEPS = 31
TOP_BITS = 0x7F800000


def _topk_mask_kernel(a_ref, o_ref):
    x = a_ref[...]

    lo0 = jnp.zeros((BLOCK_ROWS, 1), jnp.int32)
    hi0 = jnp.full((BLOCK_ROWS, 1), TOP_BITS, jnp.int32)
    cnt0 = jnp.full((BLOCK_ROWS, 1), N, jnp.int32)

    def cond(carry):
        i, lo, hi, cnt_lo = carry
        settled = jnp.logical_or(cnt_lo == K, hi - lo <= 1)
        return jnp.logical_and(i < BITS_STEPS, jnp.logical_not(jnp.all(settled)))

    def body(carry):
        i, lo, hi, cnt_lo = carry
        mid = lo + lax.shift_right_logical(hi - lo, 1)
        mid_f = lax.bitcast_convert_type(mid, jnp.float32)
        cnt = jnp.sum((x >= mid_f).astype(jnp.int32), axis=1, keepdims=True)
        take_hi = cnt >= K
        lo = jnp.where(take_hi, mid, lo)
        hi = jnp.where(take_hi, hi, mid)
        cnt_lo = jnp.where(take_hi, cnt, cnt_lo)
        return i + 1, lo, hi, cnt_lo

    _, lo, _, _ = lax.while_loop(cond, body, (0, lo0, hi0, cnt0))
    lo_f = lax.bitcast_convert_type(lo, jnp.float32)
    o_ref[...] = jnp.where(x >= lo_f, x, 0.0)


@jax.jit
def _topk_mask(a):
    grid = (N // BLOCK_ROWS,)
    return pl.pallas_call(
        _topk_mask_kernel,
        grid=grid,
        in_specs=[pl.BlockSpec((BLOCK_ROWS, N), lambda i: (i, 0))],
        out_specs=pl.BlockSpec((BLOCK_ROWS, N), lambda i: (i, 0)),
        out_shape=jax.ShapeDtypeStruct((N, N), jnp.float32),
    )(a)


def kernel(idx, A_param):
    del idx
    return _topk_mask(A_param)
